# baseline (speedup 1.0000x reference)
import numpy as np

import jax
import jax.numpy as jnp
from jax.experimental import pallas as pl
from jax.experimental.pallas import tpu as pltpu

N = 4096
ROWS = 512
NPANEL = N // ROWS
NP1 = NPANEL
NP2 = NPANEL * (NPANEL - 1) // 2
NSTEPS = NP1 + NP2


def _schedule():
    ii = np.zeros(NSTEPS, dtype=np.int32)
    jj = np.zeros(NSTEPS, dtype=np.int32)
    t = NP1
    for i in range(NPANEL - 1):
        for j in range(i + 1, NPANEL):
            ii[t], jj[t] = i, j
            t += 1
    return ii, jj


_II, _JJ = _schedule()


def _body(ii_ref, jj_ref, x_ref, w1_ref, b1_ref, w2_ref, b2_ref,
          a_panel_ref, a_block_ref, out_ref, s1_scr, s2_scr, part_scr):
    t = pl.program_id(0)

    @pl.when(t == 0)
    def _():
        s1_scr[...] = jnp.dot(x_ref[...], w1_ref[...],
                              preferred_element_type=jnp.float32)

    @pl.when(t < NP1)
    def _phase1():
        i = t
        h = jnp.dot(a_panel_ref[...], s1_scr[...],
                    preferred_element_type=jnp.float32)
        h = jnp.maximum(h + b1_ref[...], 0.0)
        s2_i = jnp.dot(h, w2_ref[...], preferred_element_type=jnp.float32)
        s2_scr[pl.ds(i * ROWS, ROWS), :] = s2_i
        row_id = jax.lax.broadcasted_iota(jnp.int32, (N, 1), 0)
        s2m = jnp.where(row_id < (i + 1) * ROWS, s2_scr[...], 0.0)
        partial = jnp.dot(a_panel_ref[...], s2m,
                          preferred_element_type=jnp.float32)

        @pl.when(i < NPANEL - 1)
        def _():
            part_scr[pl.ds(i * ROWS, ROWS), :] = partial

        @pl.when(i == NPANEL - 1)
        def _():
            out_ref[...] = partial + b2_ref[...]

    @pl.when(t >= NP1)
    def _phase2():
        i = ii_ref[t]
        j = jj_ref[t]
        contrib = jnp.dot(a_block_ref[...], s2_scr[pl.ds(j * ROWS, ROWS), :],
                          preferred_element_type=jnp.float32)

        @pl.when(j == i + 1)
        def _():
            out_ref[...] = (part_scr[pl.ds(i * ROWS, ROWS), :]
                            + b2_ref[...] + contrib)

        @pl.when(j > i + 1)
        def _():
            out_ref[...] += contrib


def _gcn_branch(adj, x, w1, b1, w2, b2):
    f_in = x.shape[1]
    h1 = w1.shape[1]
    h2 = w2.shape[1]
    b1r = b1.reshape(1, h1)
    b2r = b2.reshape(1, h2)
    ii = jnp.asarray(_II)
    jj = jnp.asarray(_JJ)

    grid_spec = pltpu.PrefetchScalarGridSpec(
        num_scalar_prefetch=2,
        grid=(NSTEPS,),
        in_specs=[
            pl.BlockSpec((N, f_in), lambda t, ii, jj: (0, 0)),
            pl.BlockSpec((f_in, h1), lambda t, ii, jj: (0, 0)),
            pl.BlockSpec((1, h1), lambda t, ii, jj: (0, 0)),
            pl.BlockSpec((h1, h2), lambda t, ii, jj: (0, 0)),
            pl.BlockSpec((1, h2), lambda t, ii, jj: (0, 0)),
            pl.BlockSpec((ROWS, N),
                         lambda t, ii, jj: (jnp.minimum(t, NP1 - 1), 0)),
            pl.BlockSpec((ROWS, ROWS),
                         lambda t, ii, jj: (jnp.where(t < NP1, 0, ii[t]),
                                            jnp.where(t < NP1, 0, jj[t]))),
        ],
        out_specs=pl.BlockSpec(
            (ROWS, h2),
            lambda t, ii, jj: (jnp.where(t < NP1, NPANEL - 1, ii[t]), 0)),
        scratch_shapes=[
            pltpu.VMEM((N, h1), jnp.float32),
            pltpu.VMEM((N, h2), jnp.float32),
            pltpu.VMEM((N, h2), jnp.float32),
        ],
    )

    return pl.pallas_call(
        _body,
        grid_spec=grid_spec,
        out_shape=jax.ShapeDtypeStruct((N, h2), jnp.float32),
    )(ii, jj, x, w1, b1r, w2, b2r, adj, adj)


def kernel(drug_graph, drug_sim_feat, dis_graph, disease_sim_feat,
           W1_drug, b1_drug, W2_drug, b2_drug,
           W1_dis, b1_dis, W2_dis, b2_dis):
    emb1 = _gcn_branch(drug_graph, drug_sim_feat, W1_drug, b1_drug,
                       W2_drug, b2_drug)
    emb2 = _gcn_branch(dis_graph, disease_sim_feat, W1_dis, b1_dis,
                       W2_dis, b2_dis)
    return (emb1, emb2)


# speedup vs baseline: 1.3219x; 1.3219x over previous
import numpy as np

import jax
import jax.numpy as jnp
from jax.experimental import pallas as pl
from jax.experimental.pallas import tpu as pltpu

N = 4096
ROWS = 512
NPANEL = N // ROWS
NCACHE = 4
NP1 = NPANEL
NPC = NCACHE
_PAIRS = [(i, j) for i in range(NCACHE, NPANEL - 1)
          for j in range(i + 1, NPANEL)]
NP2 = len(_PAIRS)
NSTEPS = NP1 + NPC + NP2


def _schedule():
    bi = np.full(NSTEPS, NCACHE, dtype=np.int32)
    bj = np.full(NSTEPS, NCACHE + 1, dtype=np.int32)
    for t, (i, j) in enumerate(_PAIRS):
        bi[NP1 + NPC + t] = i
        bj[NP1 + NPC + t] = j
    return bi, bj


_BI, _BJ = _schedule()


def _body(bi_ref, bj_ref, x_ref, w1_ref, b1_ref, w2_ref, b2_ref,
          cache_ref, a_panel_ref, a_block_ref, out_ref,
          s1_scr, s2_scr, part_scr):
    t = pl.program_id(0)

    @pl.when(t == 0)
    def _():
        s1_scr[...] = jnp.dot(x_ref[...], w1_ref[...],
                              preferred_element_type=jnp.float32)

    @pl.when(t < NCACHE)
    def _phase1_cached():
        a = cache_ref[pl.ds(t * ROWS, ROWS), :]
        h = jnp.dot(a, s1_scr[...], preferred_element_type=jnp.float32)
        h = jnp.maximum(h + b1_ref[...], 0.0)
        s2_scr[pl.ds(t * ROWS, ROWS), :] = jnp.dot(
            h, w2_ref[...], preferred_element_type=jnp.float32)

    @pl.when(jnp.logical_and(t >= NCACHE, t < NP1))
    def _phase1_streamed():
        a = a_panel_ref[...]
        h = jnp.dot(a, s1_scr[...], preferred_element_type=jnp.float32)
        h = jnp.maximum(h + b1_ref[...], 0.0)
        s2_scr[pl.ds(t * ROWS, ROWS), :] = jnp.dot(
            h, w2_ref[...], preferred_element_type=jnp.float32)
        row_id = jax.lax.broadcasted_iota(jnp.int32, (N, 1), 0)
        s2m = jnp.where(row_id < (t + 1) * ROWS, s2_scr[...], 0.0)
        partial = jnp.dot(a, s2m, preferred_element_type=jnp.float32)

        @pl.when(t < NPANEL - 1)
        def _():
            part_scr[pl.ds(t * ROWS, ROWS), :] = partial

        @pl.when(t == NPANEL - 1)
        def _():
            out_ref[...] = partial + b2_ref[...]

    @pl.when(jnp.logical_and(t >= NP1, t < NP1 + NPC))
    def _out_cached():
        i = t - NP1
        a = cache_ref[pl.ds(i * ROWS, ROWS), :]
        out_ref[...] = jnp.dot(
            a, s2_scr[...], preferred_element_type=jnp.float32) + b2_ref[...]

    @pl.when(t >= NP1 + NPC)
    def _phase2():
        i = bi_ref[t]
        j = bj_ref[t]
        contrib = jnp.dot(a_block_ref[...], s2_scr[pl.ds(j * ROWS, ROWS), :],
                          preferred_element_type=jnp.float32)

        @pl.when(j == i + 1)
        def _():
            out_ref[...] = (part_scr[pl.ds(i * ROWS, ROWS), :]
                            + b2_ref[...] + contrib)

        @pl.when(j > i + 1)
        def _():
            out_ref[...] += contrib


def _gcn_branch(adj, x, w1, b1, w2, b2):
    f_in = x.shape[1]
    h1 = w1.shape[1]
    h2 = w2.shape[1]
    b1r = b1.reshape(1, h1)
    b2r = b2.reshape(1, h2)
    bi = jnp.asarray(_BI)
    bj = jnp.asarray(_BJ)

    grid_spec = pltpu.PrefetchScalarGridSpec(
        num_scalar_prefetch=2,
        grid=(NSTEPS,),
        in_specs=[
            pl.BlockSpec((N, f_in), lambda t, bi, bj: (0, 0)),
            pl.BlockSpec((f_in, h1), lambda t, bi, bj: (0, 0)),
            pl.BlockSpec((1, h1), lambda t, bi, bj: (0, 0)),
            pl.BlockSpec((h1, h2), lambda t, bi, bj: (0, 0)),
            pl.BlockSpec((1, h2), lambda t, bi, bj: (0, 0)),
            pl.BlockSpec((NCACHE * ROWS, N), lambda t, bi, bj: (0, 0)),
            pl.BlockSpec((ROWS, N),
                         lambda t, bi, bj: (jnp.clip(t, NCACHE, NPANEL - 1),
                                            0)),
            pl.BlockSpec((ROWS, ROWS),
                         lambda t, bi, bj: (bi[t], bj[t])),
        ],
        out_specs=pl.BlockSpec(
            (ROWS, h2),
            lambda t, bi, bj: (
                jnp.where(t < NP1, NPANEL - 1,
                          jnp.where(t < NP1 + NPC, t - NP1, bi[t])), 0)),
        scratch_shapes=[
            pltpu.VMEM((N, h1), jnp.float32),
            pltpu.VMEM((N, h2), jnp.float32),
            pltpu.VMEM((N, h2), jnp.float32),
        ],
    )

    return pl.pallas_call(
        _body,
        grid_spec=grid_spec,
        out_shape=jax.ShapeDtypeStruct((N, h2), jnp.float32),
        compiler_params=pltpu.CompilerParams(
            vmem_limit_bytes=64 * 1024 * 1024),
    )(bi, bj, x, w1, b1r, w2, b2r, adj, adj, adj)


def kernel(drug_graph, drug_sim_feat, dis_graph, disease_sim_feat,
           W1_drug, b1_drug, W2_drug, b2_drug,
           W1_dis, b1_dis, W2_dis, b2_dis):
    emb1 = _gcn_branch(drug_graph, drug_sim_feat, W1_drug, b1_drug,
                       W2_drug, b2_drug)
    emb2 = _gcn_branch(dis_graph, disease_sim_feat, W1_dis, b1_dis,
                       W2_dis, b2_dis)
    return (emb1, emb2)


# speedup vs baseline: 1.8171x; 1.3747x over previous
import jax
import jax.numpy as jnp
from jax.experimental import pallas as pl
from jax.experimental.pallas import tpu as pltpu

N = 4096
ROWS = 512
NPANEL = N // ROWS
NSTEPS = 2 * NPANEL


def _body(x_ref, w1_ref, b1_ref, w2_ref, b2_ref, a_panel_ref, out_ref,
          abf_scr, s1_scr, s2_scr):
    t = pl.program_id(0)

    @pl.when(t == 0)
    def _():
        s1 = jnp.dot(x_ref[...], w1_ref[...],
                     preferred_element_type=jnp.float32)
        s1_scr[...] = s1.astype(jnp.bfloat16)

    @pl.when(t < NPANEL)
    def _pass1():
        a_bf = a_panel_ref[...].astype(jnp.bfloat16)
        abf_scr[pl.ds(t * ROWS, ROWS), :] = a_bf
        h = jnp.dot(a_bf, s1_scr[...], preferred_element_type=jnp.float32)
        h = jnp.maximum(h + b1_ref[...], 0.0)
        s2 = jnp.dot(h, w2_ref[...], preferred_element_type=jnp.float32)
        s2_scr[pl.ds(t * ROWS, ROWS), :] = s2.astype(jnp.bfloat16)

    @pl.when(t >= NPANEL)
    def _pass2():
        i = t - NPANEL
        out_ref[...] = jnp.dot(abf_scr[pl.ds(i * ROWS, ROWS), :],
                               s2_scr[...],
                               preferred_element_type=jnp.float32) + b2_ref[...]


def _gcn_branch(adj, x, w1, b1, w2, b2):
    f_in = x.shape[1]
    h1 = w1.shape[1]
    h2 = w2.shape[1]
    b1r = b1.reshape(1, h1)
    b2r = b2.reshape(1, h2)

    return pl.pallas_call(
        _body,
        grid=(NSTEPS,),
        in_specs=[
            pl.BlockSpec((N, f_in), lambda t: (0, 0)),
            pl.BlockSpec((f_in, h1), lambda t: (0, 0)),
            pl.BlockSpec((1, h1), lambda t: (0, 0)),
            pl.BlockSpec((h1, h2), lambda t: (0, 0)),
            pl.BlockSpec((1, h2), lambda t: (0, 0)),
            pl.BlockSpec((ROWS, N),
                         lambda t: (jnp.minimum(t, NPANEL - 1), 0)),
        ],
        out_specs=pl.BlockSpec(
            (ROWS, h2), lambda t: (jnp.maximum(t - NPANEL, 0), 0)),
        out_shape=jax.ShapeDtypeStruct((N, h2), jnp.float32),
        scratch_shapes=[
            pltpu.VMEM((N, N), jnp.bfloat16),
            pltpu.VMEM((N, h1), jnp.bfloat16),
            pltpu.VMEM((N, h2), jnp.bfloat16),
        ],
        compiler_params=pltpu.CompilerParams(
            vmem_limit_bytes=100 * 1024 * 1024),
    )(x, w1, b1r, w2, b2r, adj)


def kernel(drug_graph, drug_sim_feat, dis_graph, disease_sim_feat,
           W1_drug, b1_drug, W2_drug, b2_drug,
           W1_dis, b1_dis, W2_dis, b2_dis):
    emb1 = _gcn_branch(drug_graph, drug_sim_feat, W1_drug, b1_drug,
                       W2_drug, b2_drug)
    emb2 = _gcn_branch(dis_graph, disease_sim_feat, W1_dis, b1_dis,
                       W2_dis, b2_dis)
    return (emb1, emb2)


# speedup vs baseline: 1.8411x; 1.0132x over previous
import jax
import jax.numpy as jnp
from jax.experimental import pallas as pl
from jax.experimental.pallas import tpu as pltpu

N = 4096
ROWS = 256
NP = N // ROWS
OROWS = 1024
NO2 = N // OROWS
NSTEPS = 2 * NP + NO2


def _body(x1_ref, w11_ref, b11_ref, w12_ref, b12_ref,
          x2_ref, w21_ref, b21_ref, w22_ref, b22_ref,
          a1_ref, a2_ref, out1_ref, out2_ref,
          abf_scr, s11_scr, s12_scr, s21_scr, s22_scr):
    t = pl.program_id(0)

    @pl.when(t == 0)
    def _():
        s11_scr[...] = jnp.dot(
            x1_ref[...], w11_ref[...],
            preferred_element_type=jnp.float32).astype(jnp.bfloat16)
        s12_scr[...] = jnp.dot(
            x2_ref[...], w21_ref[...],
            preferred_element_type=jnp.float32).astype(jnp.bfloat16)

    @pl.when(t < NP)
    def _b1_pass1():
        a_bf = a1_ref[...].astype(jnp.bfloat16)
        abf_scr[pl.ds(t * ROWS, ROWS), :] = a_bf
        h = jnp.dot(a_bf, s11_scr[...], preferred_element_type=jnp.float32)
        h = jnp.maximum(h + b11_ref[...], 0.0)
        s2 = jnp.dot(h, w12_ref[...], preferred_element_type=jnp.float32)
        s21_scr[pl.ds(t * ROWS, ROWS), :] = s2.astype(jnp.bfloat16)

    @pl.when(jnp.logical_and(t >= NP, t < 2 * NP))
    def _b1_pass2_b2_pass1():
        i = t - NP
        out1_ref[...] = jnp.dot(
            abf_scr[pl.ds(i * ROWS, ROWS), :], s21_scr[...],
            preferred_element_type=jnp.float32) + b12_ref[...]
        a_bf = a2_ref[...].astype(jnp.bfloat16)
        abf_scr[pl.ds(i * ROWS, ROWS), :] = a_bf
        h = jnp.dot(a_bf, s12_scr[...], preferred_element_type=jnp.float32)
        h = jnp.maximum(h + b21_ref[...], 0.0)
        s2 = jnp.dot(h, w22_ref[...], preferred_element_type=jnp.float32)
        s22_scr[pl.ds(i * ROWS, ROWS), :] = s2.astype(jnp.bfloat16)

    @pl.when(t >= 2 * NP)
    def _b2_pass2():
        j = t - 2 * NP
        out2_ref[...] = jnp.dot(
            abf_scr[pl.ds(j * OROWS, OROWS), :], s22_scr[...],
            preferred_element_type=jnp.float32) + b22_ref[...]


def _net(adj1, x1, w11, b11, w12, b12, adj2, x2, w21, b21, w22, b22):
    f1 = x1.shape[1]
    f2 = x2.shape[1]
    h1 = w11.shape[1]
    h2 = w12.shape[1]

    return pl.pallas_call(
        _body,
        grid=(NSTEPS,),
        in_specs=[
            pl.BlockSpec((N, f1), lambda t: (0, 0)),
            pl.BlockSpec((f1, h1), lambda t: (0, 0)),
            pl.BlockSpec((1, h1), lambda t: (0, 0)),
            pl.BlockSpec((h1, h2), lambda t: (0, 0)),
            pl.BlockSpec((1, h2), lambda t: (0, 0)),
            pl.BlockSpec((N, f2), lambda t: (0, 0)),
            pl.BlockSpec((f2, h1), lambda t: (0, 0)),
            pl.BlockSpec((1, h1), lambda t: (0, 0)),
            pl.BlockSpec((h1, h2), lambda t: (0, 0)),
            pl.BlockSpec((1, h2), lambda t: (0, 0)),
            pl.BlockSpec((ROWS, N),
                         lambda t: (jnp.clip(t, 0, NP - 1), 0)),
            pl.BlockSpec((ROWS, N),
                         lambda t: (jnp.clip(t - NP, 0, NP - 1), 0)),
        ],
        out_specs=[
            pl.BlockSpec((ROWS, h2),
                         lambda t: (jnp.clip(t - NP, 0, NP - 1), 0)),
            pl.BlockSpec((OROWS, h2),
                         lambda t: (jnp.clip(t - 2 * NP, 0, NO2 - 1), 0)),
        ],
        out_shape=[
            jax.ShapeDtypeStruct((N, h2), jnp.float32),
            jax.ShapeDtypeStruct((N, h2), jnp.float32),
        ],
        scratch_shapes=[
            pltpu.VMEM((N, N), jnp.bfloat16),
            pltpu.VMEM((N, h1), jnp.bfloat16),
            pltpu.VMEM((N, h1), jnp.bfloat16),
            pltpu.VMEM((N, h2), jnp.bfloat16),
            pltpu.VMEM((N, h2), jnp.bfloat16),
        ],
        compiler_params=pltpu.CompilerParams(
            vmem_limit_bytes=64 * 1024 * 1024),
    )(x1, w11, b11.reshape(1, h1), w12, b12.reshape(1, h2),
      x2, w21, b21.reshape(1, h1), w22, b22.reshape(1, h2),
      adj1, adj2)


def kernel(drug_graph, drug_sim_feat, dis_graph, disease_sim_feat,
           W1_drug, b1_drug, W2_drug, b2_drug,
           W1_dis, b1_dis, W2_dis, b2_dis):
    emb1, emb2 = _net(drug_graph, drug_sim_feat, W1_drug, b1_drug,
                      W2_drug, b2_drug,
                      dis_graph, disease_sim_feat, W1_dis, b1_dis,
                      W2_dis, b2_dis)
    return (emb1, emb2)
